# baseline (speedup 1.0000x reference)
import functools

import jax
import jax.numpy as jnp
from jax import lax
from jax.experimental import pallas as pl
from jax.experimental.pallas import tpu as pltpu
from jax.experimental.pallas import tpu_sc as plsc

B, S, D, E = 4, 2048, 4096, 64

S_TC = 1792
S_SC = S - S_TC

TC_CHUNK = 128
TC_STEPS = S_TC // TC_CHUNK

NC, NSUB, LANES = 2, 16, 16
NW = NC * NSUB
SPLIT = NW // B
SLAB = S_SC // SPLIT
ROWS = 8
NCHUNK = SLAB // ROWS
STRIP = 16

_mesh = plsc.VectorSubcoreMesh(core_axis_name="c", subcore_axis_name="s")


@functools.partial(
    pl.kernel,
    mesh=_mesh,
    out_type=jax.ShapeDtypeStruct((SPLIT, B, D), jnp.float32),
    scratch_types=[
        pltpu.VMEM((ROWS, D), jnp.float32),
        pltpu.VMEM((ROWS, D), jnp.float32),
        pltpu.VMEM((D,), jnp.float32),
        pltpu.SemaphoreType.DMA,
        pltpu.SemaphoreType.DMA,
    ],
)
def _pool_sc(h_hbm, out_hbm, buf0, buf1, acc, sem0, sem1):
    wid = lax.axis_index("s") * NC + lax.axis_index("c")
    b = wid // SPLIT
    j = wid % SPLIT
    s_base = S_TC + j * SLAB

    for k in range(D // LANES):
        acc[pl.ds(k * LANES, LANES)] = jnp.zeros((LANES,), jnp.float32)

    def _copy(g, buf, sem):
        return pltpu.make_async_copy(
            h_hbm.at[b, pl.ds(s_base + g * ROWS, ROWS)], buf, sem)

    def _accum(buf):
        for strip in range(D // (STRIP * LANES)):
            base = strip * STRIP * LANES
            accs = [buf[0, pl.ds(base + k * LANES, LANES)]
                    for k in range(STRIP)]
            for r in range(1, ROWS):
                for k in range(STRIP):
                    accs[k] = accs[k] + buf[r, pl.ds(base + k * LANES, LANES)]
            for k in range(STRIP):
                plsc.addupdate(acc.at[pl.ds(base + k * LANES, LANES)], accs[k])

    _copy(0, buf0, sem0).start()

    def _body(i, carry):
        g = 2 * i
        _copy(g + 1, buf1, sem1).start()
        _copy(g, buf0, sem0).wait()
        _accum(buf0)

        @pl.when(g + 2 < NCHUNK)
        def _():
            _copy(g + 2, buf0, sem0).start()

        _copy(g + 1, buf1, sem1).wait()
        _accum(buf1)
        return carry

    lax.fori_loop(0, NCHUNK // 2, _body, 0)
    pltpu.sync_copy(acc, out_hbm.at[j, b])


def _pool_tc_body(h_ref, sums_ref, acc_ref):
    i = pl.program_id(0)

    @pl.when(i == 0)
    def _init():
        acc_ref[...] = jnp.zeros_like(acc_ref)

    acc_ref[...] += jnp.sum(h_ref[...], axis=1)

    @pl.when(i == TC_STEPS - 1)
    def _out():
        sums_ref[...] = acc_ref[...]


def _finish_body(tc_ref, sc_ref, w_ref, oh_ref, logits_ref, probs_ref,
                 loss_ref):
    pooled = (tc_ref[...] + jnp.sum(sc_ref[...], axis=0)) * (1.0 / S)
    logits = jax.lax.dot_general(
        pooled, w_ref[...], (((1,), (1,)), ((), ())),
        preferred_element_type=jnp.float32)
    m = jnp.max(logits, axis=1, keepdims=True)
    ex = jnp.exp(logits - m)
    se = jnp.sum(ex, axis=1, keepdims=True)
    logits_ref[...] = logits
    probs_ref[...] = ex / se
    lse = jnp.log(se) + m
    picked = jnp.sum(logits * oh_ref[...], axis=1, keepdims=True)
    loss_ref[...] = jnp.mean(lse - picked).reshape(1, 1)


@jax.jit
def kernel(hidden_states, W, task_labels):
    onehot = (task_labels[:, None] == jnp.arange(E, dtype=jnp.int32)[None, :])
    onehot = onehot.astype(jnp.float32)
    sc_sums = _pool_sc(hidden_states)
    tc_sums = pl.pallas_call(
        _pool_tc_body,
        grid=(TC_STEPS,),
        in_specs=[pl.BlockSpec((B, TC_CHUNK, D), lambda i: (0, i, 0))],
        out_specs=pl.BlockSpec((B, D), lambda i: (0, 0)),
        out_shape=jax.ShapeDtypeStruct((B, D), jnp.float32),
        scratch_shapes=[pltpu.VMEM((B, D), jnp.float32)],
    )(hidden_states)
    logits, probs, loss = pl.pallas_call(
        _finish_body,
        out_shape=[
            jax.ShapeDtypeStruct((B, E), jnp.float32),
            jax.ShapeDtypeStruct((B, E), jnp.float32),
            jax.ShapeDtypeStruct((1, 1), jnp.float32),
        ],
    )(tc_sums, sc_sums, W, onehot)
    return logits, probs, loss.reshape(())


# speedup vs baseline: 1.4419x; 1.4419x over previous
import jax
import jax.numpy as jnp
from jax.experimental import pallas as pl
from jax.experimental.pallas import tpu as pltpu

B, S, D, E = 4, 2048, 4096, 64
S_CHUNK = 128
NS = S // S_CHUNK


def _router_body(lab_ref, h_ref, w_ref, logits_ref, probs_ref, loss_ref,
                 acc_ref):
    i = pl.program_id(0)

    @pl.when(i == 0)
    def _init():
        acc_ref[...] = jnp.zeros_like(acc_ref)

    acc_ref[...] += jnp.sum(h_ref[...], axis=1)

    @pl.when(i == NS - 1)
    def _epilogue():
        pooled = acc_ref[...] * (1.0 / S)
        logits = jax.lax.dot_general(
            pooled, w_ref[...], (((1,), (1,)), ((), ())),
            preferred_element_type=jnp.float32)
        m = jnp.max(logits, axis=1, keepdims=True)
        ex = jnp.exp(logits - m)
        se = jnp.sum(ex, axis=1, keepdims=True)
        logits_ref[...] = logits
        probs_ref[...] = ex / se
        lse = jnp.log(se) + m
        labcol = jnp.concatenate(
            [jnp.full((1, E), lab_ref[0, b], jnp.int32) for b in range(B)],
            axis=0)
        onehot = (labcol == jax.lax.broadcasted_iota(jnp.int32, (B, E), 1))
        picked = jnp.sum(jnp.where(onehot, logits, 0.0), axis=1,
                         keepdims=True)
        loss_ref[...] = jnp.mean(lse - picked).reshape(1, 1)


@jax.jit
def kernel(hidden_states, W, task_labels):
    logits, probs, loss = pl.pallas_call(
        _router_body,
        grid=(NS,),
        in_specs=[
            pl.BlockSpec(memory_space=pltpu.SMEM),
            pl.BlockSpec((B, S_CHUNK, D), lambda i: (0, i, 0)),
            pl.BlockSpec((E, D), lambda i: (0, 0)),
        ],
        out_specs=[
            pl.BlockSpec((B, E), lambda i: (0, 0)),
            pl.BlockSpec((B, E), lambda i: (0, 0)),
            pl.BlockSpec((1, 1), lambda i: (0, 0)),
        ],
        out_shape=[
            jax.ShapeDtypeStruct((B, E), jnp.float32),
            jax.ShapeDtypeStruct((B, E), jnp.float32),
            jax.ShapeDtypeStruct((1, 1), jnp.float32),
        ],
        scratch_shapes=[pltpu.VMEM((B, D), jnp.float32)],
    )(task_labels.reshape(1, B), hidden_states, W)
    return logits, probs, loss.reshape(())
